# baseline (speedup 1.0000x reference)
import functools

import jax
import jax.numpy as jnp
from jax import lax
from jax.experimental import pallas as pl
from jax.experimental.pallas import tpu as pltpu
from jax.experimental.pallas import tpu_sc as plsc

N = 10000
E = 320000
D = 128

NC = 2
NS = 16
NW = NC * NS
BLK = 128
EB = 80
EP = NW * BLK * EB
EPW = EP // NW
EPH = EP // NC
NBLK = EPH // BLK
WBLK = NBLK // NS
NBUF = 4
NGRP = WBLK // NBUF
NTL = WBLK - NGRP * NBUF
N_PAD = 10240
RPT = N_PAD // NS
HOLE = RPT
HR = 4
RB = 1024
NRB = N_PAD // RB
ZB = N_PAD // BLK
ZPT = ZB // NS

_mesh = plsc.VectorSubcoreMesh(core_axis_name="c", subcore_axis_name="s")


@functools.partial(
    pl.kernel,
    out_type=(jax.ShapeDtypeStruct((NW, N_PAD), jnp.float32),
              jax.ShapeDtypeStruct((NW, N_PAD), jnp.float32)),
    mesh=_mesh,
    compiler_params=pltpu.CompilerParams(needs_layout_passes=False),
    scratch_types=(
        pltpu.VMEM((EPW,), jnp.int32),
        pltpu.VMEM((EPW,), jnp.int32),
        pltpu.VMEM((N_PAD,), jnp.float32),
        pltpu.VMEM((N_PAD,), jnp.float32),
    ),
)
def _degrees(src_hbm, dst_hbm, zn_hbm, dego_out, degi_out,
             src_v, dst_v, ho_v, hi_v):
    c = lax.axis_index("c")
    s = lax.axis_index("s")
    wid = s * NC + c
    pltpu.sync_copy(zn_hbm, ho_v)
    pltpu.sync_copy(zn_hbm, hi_v)
    pltpu.sync_copy(src_hbm.at[pl.ds(wid * EPW, EPW)], src_v)
    pltpu.sync_copy(dst_hbm.at[pl.ds(wid * EPW, EPW)], dst_v)
    ones = jnp.full((16,), 1.0, jnp.float32)

    def body(i, carry):
        sv = src_v[pl.ds(i * 16, 16)]
        dv = dst_v[pl.ds(i * 16, 16)]
        plsc.addupdate_scatter(ho_v, [sv], ones)
        plsc.addupdate_scatter(hi_v, [dv], ones)
        return carry

    lax.fori_loop(0, EPW // 16, body, 0)
    pltpu.sync_copy(ho_v, dego_out.at[wid])
    pltpu.sync_copy(hi_v, degi_out.at[wid])


@functools.partial(
    pl.kernel,
    out_type=jax.ShapeDtypeStruct((NC * N_PAD, 1, D), jnp.float32),
    mesh=_mesh,
    scratch_types=(
        pltpu.VMEM((WBLK, 1, BLK), jnp.int32),
        pltpu.VMEM((WBLK, 1, BLK), jnp.int32),
        pltpu.VMEM((ZPT, 1, BLK), jnp.int32),
        pltpu.VMEM((BLK, 1, D), jnp.float32),
        pltpu.VMEM((BLK, 1, D), jnp.float32),
        pltpu.VMEM((BLK, 1, D), jnp.float32),
        pltpu.VMEM((BLK, 1, D), jnp.float32),
        pltpu.VMEM_SHARED((RPT, 1, D), jnp.float32),
        pltpu.SemaphoreType.DMA,
        pltpu.SemaphoreType.DMA,
    ),
)
def _edge_agg(h_hbm, src_hbm, dst_hbm, zrows_hbm, zidx_hbm, agg_out,
              src_v, dst_v, zidx_v, buf0, buf1, buf2, buf3, acc_sp,
              gsem, gsem1):
    c = lax.axis_index("c")
    s = lax.axis_index("s")
    bufs = (buf0, buf1, buf2, buf3)
    pltpu.sync_copy(zrows_hbm, buf0)
    pltpu.sync_copy(zidx_hbm.at[pl.ds(s * ZPT, ZPT)], zidx_v)
    for k in range(ZPT):
        pltpu.sync_copy(buf0, acc_sp.at[zidx_v.at[k, 0]])
    tile_base = c * NBLK + s * WBLK
    pltpu.sync_copy(src_hbm.at[pl.ds(tile_base, WBLK)], src_v)
    pltpu.sync_copy(dst_hbm.at[pl.ds(tile_base, WBLK)], dst_v)
    plsc.subcore_barrier()

    def blk_body(j, carry):
        pltpu.async_copy(h_hbm.at[src_v.at[j, 0]], buf0, gsem).wait()
        pltpu.sync_copy(buf0, acc_sp.at[dst_v.at[j, 0]], add=True)
        return carry

    lax.fori_loop(0, WBLK, blk_body, 0)
    plsc.subcore_barrier()

    for k in range(ZPT):
        pltpu.async_copy(acc_sp.at[zidx_v.at[k, 0]], buf0, gsem).wait()
        base = c * N_PAD + (s * ZPT + k) * BLK
        pltpu.sync_copy(buf0, agg_out.at[pl.ds(base, BLK)])


def _norm_from_hist(deg_ref):
    deg = jnp.sum(deg_ref[...], axis=0)
    return lax.rsqrt(jnp.maximum(deg, 1.0))[:, None]


def _prep_body(feat_ref, dego_ref, out_ref):
    out_ref[...] = feat_ref[...] * _norm_from_hist(dego_ref)


def _dense1_body(aggp_ref, degi_ref, dego_ref, w_ref, b_ref, a_ref, out_ref):
    agg = (aggp_ref[0] + aggp_ref[1]) * _norm_from_hist(degi_ref)
    t = jnp.dot(agg, w_ref[...], preferred_element_type=jnp.float32) + b_ref[...]
    h = jnp.where(t >= 0, t, a_ref[...] * t)
    out_ref[...] = h * _norm_from_hist(dego_ref)


def _dense2_body(aggp_ref, degi_ref, w_ref, b_ref, out_ref):
    agg = (aggp_ref[0] + aggp_ref[1]) * _norm_from_hist(degi_ref)
    out_ref[...] = (jnp.dot(agg, w_ref[...], preferred_element_type=jnp.float32)
                    + b_ref[...])


_row_spec = pl.BlockSpec((RB, D), lambda i: (i, 0))
_deg_spec = pl.BlockSpec((NW, RB), lambda i: (0, i))
_agg_spec = pl.BlockSpec((NC, RB, D), lambda i: (0, i, 0))
_w_spec = pl.BlockSpec((D, D), lambda i: (0, 0))
_v_spec = pl.BlockSpec((D,), lambda i: (0,))
_out_sds = jax.ShapeDtypeStruct((N_PAD, D), jnp.float32)
_grid = (NRB,)

_prep = pl.pallas_call(
    _prep_body, grid=_grid,
    in_specs=[_row_spec, _deg_spec],
    out_specs=_row_spec, out_shape=_out_sds)

_dense1 = pl.pallas_call(
    _dense1_body, grid=_grid,
    in_specs=[_agg_spec, _deg_spec, _deg_spec, _w_spec, _v_spec, _v_spec],
    out_specs=_row_spec, out_shape=_out_sds)

_dense2 = pl.pallas_call(
    _dense2_body, grid=_grid,
    in_specs=[_agg_spec, _deg_spec, _w_spec, _v_spec],
    out_specs=_row_spec, out_shape=_out_sds)


def _t3(h):
    return h.reshape(N_PAD, 1, D)


def kernel(feat, edge_index, W1, b1, a1, W2, b2):
    src = edge_index[0]
    dst = edge_index[1]
    pad = EP - E
    srcp = jnp.concatenate([src, jnp.full((pad,), N, jnp.int32)])
    dstp = jnp.concatenate([dst, jnp.full((pad,), N, jnp.int32)])
    srcp = srcp + HR * (srcp >= HOLE).astype(jnp.int32)
    dstp = dstp + HR * (dstp >= HOLE).astype(jnp.int32)
    srcb = srcp.reshape(NC * NBLK, 1, BLK)
    dstb = dstp.reshape(NC * NBLK, 1, BLK)
    feat_pad = jnp.concatenate(
        [feat[:HOLE], jnp.zeros((HR, D), feat.dtype), feat[HOLE:],
         jnp.zeros((N_PAD - N - HR, D), feat.dtype)], axis=0)
    zeros_n = jnp.zeros((N_PAD,), jnp.float32)
    zrows = jnp.zeros((BLK, 1, D), jnp.float32)
    zidx = jnp.arange(N_PAD, dtype=jnp.int32).reshape(ZB, 1, BLK)

    dego, degi = _degrees(srcp, dstp, zeros_n)

    h1 = _prep(feat_pad, dego)
    agg1 = _edge_agg(_t3(h1), srcb, dstb, zrows, zidx).reshape(NC, N_PAD, D)
    h2 = _dense1(agg1, degi, dego, W1, b1, a1)
    agg2 = _edge_agg(_t3(h2), srcb, dstb, zrows, zidx).reshape(NC, N_PAD, D)
    out = _dense2(agg2, degi, W2, b2)
    return jnp.concatenate([out[:HOLE], out[HOLE + HR:N + HR]], axis=0)


# speedup vs baseline: 1.4025x; 1.4025x over previous
import functools

import jax
import jax.numpy as jnp
from jax import lax
from jax.experimental import pallas as pl
from jax.experimental.pallas import tpu as pltpu
from jax.experimental.pallas import tpu_sc as plsc

N = 10000
E = 320000
D = 128

NC = 2
NS = 16
NW = NC * NS
BLK = 128
EB = -(-E // (NW * BLK))
EP = NW * BLK * EB
EPW = EP // NW
EPH = EP // NC
NBLK = EPH // BLK
WBLK = NBLK // NS
NBUF = 4
NGRP = WBLK // NBUF
NTL = WBLK - NGRP * NBUF
N_PAD = 10240
RPT = N_PAD // NS
HOLE = RPT
HR = 4
RB = 1024
NRB = N_PAD // RB
ZB = N_PAD // BLK
ZPT = ZB // NS

_mesh = plsc.VectorSubcoreMesh(core_axis_name="c", subcore_axis_name="s")


@functools.partial(
    pl.kernel,
    out_type=(jax.ShapeDtypeStruct((NW, N_PAD), jnp.float32),
              jax.ShapeDtypeStruct((NW, N_PAD), jnp.float32)),
    mesh=_mesh,
    compiler_params=pltpu.CompilerParams(needs_layout_passes=False),
    scratch_types=(
        pltpu.VMEM((EPW,), jnp.int32),
        pltpu.VMEM((EPW,), jnp.int32),
        pltpu.VMEM((N_PAD,), jnp.float32),
        pltpu.VMEM((N_PAD,), jnp.float32),
    ),
)
def _degrees(src_hbm, dst_hbm, zn_hbm, dego_out, degi_out,
             src_v, dst_v, ho_v, hi_v):
    c = lax.axis_index("c")
    s = lax.axis_index("s")
    wid = s * NC + c
    pltpu.sync_copy(zn_hbm, ho_v)
    pltpu.sync_copy(zn_hbm, hi_v)
    pltpu.sync_copy(src_hbm.at[pl.ds(wid * EPW, EPW)], src_v)
    pltpu.sync_copy(dst_hbm.at[pl.ds(wid * EPW, EPW)], dst_v)
    ones = jnp.full((16,), 1.0, jnp.float32)

    def body(i, carry):
        sv = src_v[pl.ds(i * 16, 16)]
        dv = dst_v[pl.ds(i * 16, 16)]
        plsc.addupdate_scatter(ho_v, [sv], ones)
        plsc.addupdate_scatter(hi_v, [dv], ones)
        return carry

    lax.fori_loop(0, EPW // 16, body, 0)
    pltpu.sync_copy(ho_v, dego_out.at[wid])
    pltpu.sync_copy(hi_v, degi_out.at[wid])


@functools.partial(
    pl.kernel,
    out_type=jax.ShapeDtypeStruct((NC * N_PAD, D), jnp.float32),
    mesh=_mesh,
    compiler_params=pltpu.CompilerParams(use_tc_tiling_on_sc=False),
    scratch_types=(
        pltpu.VMEM((WBLK, BLK), jnp.int32),
        pltpu.VMEM((WBLK, BLK), jnp.int32),
        pltpu.VMEM((ZPT, BLK), jnp.int32),
        pltpu.VMEM((BLK, D), jnp.float32),
        pltpu.VMEM((BLK, D), jnp.float32),
        pltpu.VMEM((BLK, D), jnp.float32),
        pltpu.VMEM((BLK, D), jnp.float32),
        pltpu.VMEM_SHARED((RPT, D), jnp.float32),
        pltpu.SemaphoreType.DMA,
        pltpu.SemaphoreType.DMA,
    ),
)
def _edge_agg(h_hbm, src_hbm, dst_hbm, zrows_hbm, zidx_hbm, agg_out,
              src_v, dst_v, zidx_v, buf0, buf1, buf2, buf3, acc_sp,
              gsem, gsem1):
    c = lax.axis_index("c")
    s = lax.axis_index("s")
    bufs = (buf0, buf1, buf2, buf3)
    pltpu.sync_copy(zrows_hbm, buf0)
    pltpu.sync_copy(zidx_hbm.at[pl.ds(s * ZPT, ZPT)], zidx_v)
    for k in range(ZPT):
        pltpu.sync_copy(buf0, acc_sp.at[zidx_v.at[k]])
    tile_base = c * NBLK + s * WBLK
    pltpu.sync_copy(src_hbm.at[pl.ds(tile_base, WBLK)], src_v)
    pltpu.sync_copy(dst_hbm.at[pl.ds(tile_base, WBLK)], dst_v)
    plsc.subcore_barrier()

    def blk_body(j, carry):
        pltpu.async_copy(h_hbm.at[src_v.at[j]], buf0, gsem).wait()
        pltpu.sync_copy(buf0, acc_sp.at[dst_v.at[j]], add=True)
        return carry

    lax.fori_loop(0, WBLK, blk_body, 0)
    plsc.subcore_barrier()

    for k in range(ZPT):
        pltpu.async_copy(acc_sp.at[zidx_v.at[k]], buf0, gsem).wait()
        base = c * N_PAD + (s * ZPT + k) * BLK
        pltpu.sync_copy(buf0, agg_out.at[pl.ds(base, BLK)])


def _norm_from_hist(deg_ref):
    deg = jnp.sum(deg_ref[...], axis=0)
    return lax.rsqrt(jnp.maximum(deg, 1.0))[:, None]


def _prep_body(feat_ref, dego_ref, out_ref):
    out_ref[...] = feat_ref[...] * _norm_from_hist(dego_ref)


def _dense1_body(aggp_ref, degi_ref, dego_ref, w_ref, b_ref, a_ref, out_ref):
    agg = (aggp_ref[0] + aggp_ref[1]) * _norm_from_hist(degi_ref)
    t = jnp.dot(agg, w_ref[...], preferred_element_type=jnp.float32) + b_ref[...]
    h = jnp.where(t >= 0, t, a_ref[...] * t)
    out_ref[...] = h * _norm_from_hist(dego_ref)


def _dense2_body(aggp_ref, degi_ref, w_ref, b_ref, out_ref):
    agg = (aggp_ref[0] + aggp_ref[1]) * _norm_from_hist(degi_ref)
    out_ref[...] = (jnp.dot(agg, w_ref[...], preferred_element_type=jnp.float32)
                    + b_ref[...])


_row_spec = pl.BlockSpec((RB, D), lambda i: (i, 0))
_deg_spec = pl.BlockSpec((NW, RB), lambda i: (0, i))
_agg_spec = pl.BlockSpec((NC, RB, D), lambda i: (0, i, 0))
_w_spec = pl.BlockSpec((D, D), lambda i: (0, 0))
_v_spec = pl.BlockSpec((D,), lambda i: (0,))
_out_sds = jax.ShapeDtypeStruct((N_PAD, D), jnp.float32)
_grid = (NRB,)

_prep = pl.pallas_call(
    _prep_body, grid=_grid,
    in_specs=[_row_spec, _deg_spec],
    out_specs=_row_spec, out_shape=_out_sds)

_dense1 = pl.pallas_call(
    _dense1_body, grid=_grid,
    in_specs=[_agg_spec, _deg_spec, _deg_spec, _w_spec, _v_spec, _v_spec],
    out_specs=_row_spec, out_shape=_out_sds)

_dense2 = pl.pallas_call(
    _dense2_body, grid=_grid,
    in_specs=[_agg_spec, _deg_spec, _w_spec, _v_spec],
    out_specs=_row_spec, out_shape=_out_sds)


def kernel(feat, edge_index, W1, b1, a1, W2, b2):
    src = edge_index[0]
    dst = edge_index[1]
    pad = EP - E
    srcp = jnp.concatenate([src, jnp.full((pad,), N, jnp.int32)])
    dstp = jnp.concatenate([dst, jnp.full((pad,), N, jnp.int32)])
    srcp = srcp + HR * (srcp >= HOLE).astype(jnp.int32)
    dstp = dstp + HR * (dstp >= HOLE).astype(jnp.int32)
    srcb = srcp.reshape(NC * NBLK, BLK)
    dstb = dstp.reshape(NC * NBLK, BLK)
    feat_pad = jnp.concatenate(
        [feat[:HOLE], jnp.zeros((HR, D), feat.dtype), feat[HOLE:],
         jnp.zeros((N_PAD - N - HR, D), feat.dtype)], axis=0)
    zeros_n = jnp.zeros((N_PAD,), jnp.float32)
    zrows = jnp.zeros((BLK, D), jnp.float32)
    zidx = jnp.arange(N_PAD, dtype=jnp.int32).reshape(ZB, BLK)

    dego, degi = _degrees(srcp, dstp, zeros_n)

    h1 = _prep(feat_pad, dego)
    agg1 = _edge_agg(h1, srcb, dstb, zrows, zidx).reshape(NC, N_PAD, D)
    h2 = _dense1(agg1, degi, dego, W1, b1, a1)
    agg2 = _edge_agg(h2, srcb, dstb, zrows, zidx).reshape(NC, N_PAD, D)
    out = _dense2(agg2, degi, W2, b2)
    return jnp.concatenate([out[:HOLE], out[HOLE + HR:N + HR]], axis=0)
